# baseline (speedup 1.0000x reference)
import functools

import jax
import jax.numpy as jnp
from jax import lax
from jax.experimental import pallas as pl
from jax.experimental.pallas import tpu as pltpu
from jax.experimental.pallas import tpu_sc as plsc

H = 128
_NC = 2
_NS = 16
_NW = _NC * _NS
_C = 125
_L = 16

_f32 = jnp.float32


def _proj_body(x_ref, wd_ref, ws_ref, td_ref, ts_ref):
    xb = x_ref[...]
    td_ref[...] = jnp.dot(xb, wd_ref[...], preferred_element_type=_f32)
    ts_ref[...] = jnp.dot(xb, ws_ref[...], preferred_element_type=_f32)


def _proj(x, wd, ws, bn=1000):
    n = x.shape[0]
    return pl.pallas_call(
        _proj_body,
        grid=(n // bn,),
        in_specs=[
            pl.BlockSpec((bn, H), lambda i: (i, 0)),
            pl.BlockSpec((H, H), lambda i: (0, 0)),
            pl.BlockSpec((H, H), lambda i: (0, 0)),
        ],
        out_specs=[
            pl.BlockSpec((bn, H), lambda i: (i, 0)),
            pl.BlockSpec((bn, H), lambda i: (i, 0)),
        ],
        out_shape=[jax.ShapeDtypeStruct((n, H), _f32)] * 2,
    )(x, wd, ws)


def _sc_gather_sum(td, ts, dstc, srcc):
    k = dstc.shape[1]
    nch = _NW * k
    mesh = plsc.VectorSubcoreMesh(core_axis_name="c", subcore_axis_name="s",
                                  num_cores=_NC, num_subcores=_NS)

    @functools.partial(
        pl.kernel,
        out_type=jax.ShapeDtypeStruct((nch, _C, H), _f32),
        mesh=mesh,
        scratch_types=[
            pltpu.VMEM((k, _C), jnp.int32),
            pltpu.VMEM((k, _C), jnp.int32),
            pltpu.VMEM((2, _C, H), _f32),
            pltpu.VMEM((2, _C, H), _f32),
            pltpu.SemaphoreType.DMA,
            pltpu.SemaphoreType.DMA,
            pltpu.SemaphoreType.DMA,
            pltpu.SemaphoreType.DMA,
        ],
    )
    def run(td_h, ts_h, di_h, si_h, out_h, di_v, si_v, bd_v, bs_v,
            sg0, sg1, sw0, sw1):
        wid = lax.axis_index("s") * _NC + lax.axis_index("c")
        pltpu.sync_copy(di_h.at[wid], di_v)
        pltpu.sync_copy(si_h.at[wid], si_v)
        sgs = (sg0, sg1)
        sws = (sw0, sw1)

        def issue(j, slot):
            pltpu.async_copy(td_h.at[di_v.at[j]], bd_v.at[slot], sgs[slot])
            pltpu.async_copy(ts_h.at[si_v.at[j]], bs_v.at[slot], sgs[slot])

        def wait_g(slot):
            pltpu.make_async_copy(out_h.at[0], bd_v.at[slot], sgs[slot]).wait()
            pltpu.make_async_copy(out_h.at[0], bs_v.at[slot], sgs[slot]).wait()

        def drain_w(slot):
            pltpu.make_async_copy(bd_v.at[slot], out_h.at[0], sws[slot]).wait()

        def add_chunk(slot):
            def rbody(r, carry):
                for c8 in range(H // _L):
                    sl = pl.ds(c8 * _L, _L)
                    plsc.addupdate(bd_v.at[slot, r, sl], bs_v[slot, r, sl])
                return carry
            lax.fori_loop(0, _C, rbody, 0)

        def step(j, slot):
            wait_g(slot)
            add_chunk(slot)
            pltpu.async_copy(bd_v.at[slot], out_h.at[wid * k + j], sws[slot])

        issue(0, 0)

        def pair(t, carry):
            j0 = 2 * t

            @pl.when(t > 0)
            def _():
                drain_w(1)

            issue(j0 + 1, 1)
            step(j0, 0)

            @pl.when(j0 + 2 < k)
            def _():
                drain_w(0)
                issue(j0 + 2, 0)

            step(j0 + 1, 1)
            return carry

        lax.fori_loop(0, k // 2, pair, 0)
        drain_w(0)
        drain_w(1)

    return run(td, ts, dstc, srcc)


def _edge_body(gsum_ref, e_ref, we_ref, w2_ref, w3_ref,
               b1_ref, b2_ref, b3_ref, g_ref, bt_ref, out_ref):
    h = (gsum_ref[...]
         + jnp.dot(e_ref[...], we_ref[...], preferred_element_type=_f32)
         + b1_ref[...])
    h = jnp.maximum(h, 0.0)
    h = jnp.maximum(
        jnp.dot(h, w2_ref[...], preferred_element_type=_f32) + b2_ref[...], 0.0)
    h = jnp.dot(h, w3_ref[...], preferred_element_type=_f32) + b3_ref[...]
    m = jnp.mean(h, axis=-1, keepdims=True)
    c = h - m
    v = jnp.mean(c * c, axis=-1, keepdims=True)
    out_ref[...] = c * lax.rsqrt(v + 1e-5) * g_ref[...] + bt_ref[...]


def _edge_mlp(gsum, e, we, w2, w3, b1, b2, b3, g, bt, be=2000):
    ne = e.shape[0]
    wspec = pl.BlockSpec((H, H), lambda i: (0, 0))
    bspec = pl.BlockSpec((1, H), lambda i: (0, 0))
    blk = pl.BlockSpec((be, H), lambda i: (i, 0))
    return pl.pallas_call(
        _edge_body,
        grid=(ne // be,),
        in_specs=[blk, blk, wspec, wspec, wspec,
                  bspec, bspec, bspec, bspec, bspec],
        out_specs=blk,
        out_shape=jax.ShapeDtypeStruct((ne, H), _f32),
    )(gsum, e, we, w2, w3, b1, b2, b3, g, bt)


def _sc_scatter(enew, dstc, zeros):
    k = dstc.shape[1]
    n = zeros.shape[0]
    rpt = n // _NS
    mesh = plsc.VectorSubcoreMesh(core_axis_name="c", subcore_axis_name="s",
                                  num_cores=_NC, num_subcores=_NS)

    @functools.partial(
        pl.kernel,
        out_type=jax.ShapeDtypeStruct((_NC, n, H), _f32),
        mesh=mesh,
        scratch_types=[
            pltpu.VMEM((k, _C), jnp.int32),
            pltpu.VMEM((2, _C, H), _f32),
            pltpu.MemorySpace.VMEM_SHARED((n, H), _f32),
            pltpu.SemaphoreType.DMA,
            pltpu.SemaphoreType.DMA,
        ],
    )
    def run(en_h, di_h, z_h, out_h, di_v, buf_v, acc_s, sl0, sl1):
        cid = lax.axis_index("c")
        sid = lax.axis_index("s")
        wid = sid * _NC + cid
        row0 = sid * rpt
        pltpu.sync_copy(z_h.at[pl.ds(row0, rpt)], acc_s.at[pl.ds(row0, rpt)])
        plsc.subcore_barrier()
        pltpu.sync_copy(di_h.at[wid], di_v)
        sls = (sl0, sl1)

        def load(j, slot):
            pltpu.async_copy(en_h.at[wid * k + j], buf_v.at[slot], sls[slot])

        def wait_l(slot):
            pltpu.make_async_copy(en_h.at[0], buf_v.at[slot], sls[slot]).wait()

        load(0, 0)

        def pair(t, carry):
            j0 = 2 * t
            load(j0 + 1, 1)
            wait_l(0)
            pltpu.sync_copy(buf_v.at[0], acc_s.at[di_v.at[j0]], add=True)

            @pl.when(j0 + 2 < k)
            def _():
                load(j0 + 2, 0)

            wait_l(1)
            pltpu.sync_copy(buf_v.at[1], acc_s.at[di_v.at[j0 + 1]], add=True)
            return carry

        lax.fori_loop(0, k // 2, pair, 0)
        plsc.subcore_barrier()
        pltpu.sync_copy(acc_s.at[pl.ds(row0, rpt)],
                        out_h.at[cid, pl.ds(row0, rpt)])

    return run(enew, dstc, zeros)


def _node_body(x_ref, p_ref, w1x_ref, w1a_ref, w2_ref, w3_ref,
               b1_ref, b2_ref, b3_ref, g_ref, bt_ref, out_ref):
    xb = x_ref[...]
    agg = p_ref[0] + p_ref[1]
    z = (jnp.dot(xb, w1x_ref[...], preferred_element_type=_f32)
         + jnp.dot(agg, w1a_ref[...], preferred_element_type=_f32)
         + b1_ref[...])
    z = jnp.maximum(z, 0.0)
    z = jnp.maximum(
        jnp.dot(z, w2_ref[...], preferred_element_type=_f32) + b2_ref[...], 0.0)
    z = jnp.dot(z, w3_ref[...], preferred_element_type=_f32) + b3_ref[...]
    m = jnp.mean(z, axis=-1, keepdims=True)
    c = z - m
    v = jnp.mean(c * c, axis=-1, keepdims=True)
    out_ref[...] = xb + c * lax.rsqrt(v + 1e-5) * g_ref[...] + bt_ref[...]


def _node_mlp(x, parts, w1x, w1a, w2, w3, b1, b2, b3, g, bt, bn=1000):
    n = x.shape[0]
    wspec = pl.BlockSpec((H, H), lambda i: (0, 0))
    bspec = pl.BlockSpec((1, H), lambda i: (0, 0))
    return pl.pallas_call(
        _node_body,
        grid=(n // bn,),
        in_specs=[
            pl.BlockSpec((bn, H), lambda i: (i, 0)),
            pl.BlockSpec((_NC, bn, H), lambda i: (0, i, 0)),
            wspec, wspec, wspec, wspec,
            bspec, bspec, bspec, bspec, bspec,
        ],
        out_specs=pl.BlockSpec((bn, H), lambda i: (i, 0)),
        out_shape=jax.ShapeDtypeStruct((n, H), _f32),
    )(x, parts, w1x, w1a, w2, w3, b1, b2, b3, g, bt)


def kernel(x, edge_index, e, ew1, eb1, ew2, eb2, ew3, eb3, eg, ebt,
           nw1, nb1, nw2, nb2, nw3, nb3, ng, nbt):
    n = x.shape[0]
    ne = e.shape[0]
    k = ne // (_NW * _C)

    wd, ws, we = ew1[0:H], ew1[H:2 * H], ew1[2 * H:3 * H]
    w1x, w1a = nw1[0:H], nw1[H:2 * H]
    r1 = lambda v: v.reshape(1, H)

    srcc = edge_index[0].reshape(_NW, k, _C)
    dstc = edge_index[1].reshape(_NW, k, _C)

    td, ts = _proj(x, wd, ws)
    gsum = _sc_gather_sum(td, ts, dstc, srcc)
    e_new = _edge_mlp(gsum.reshape(ne, H), e,
                      we, ew2, ew3, r1(eb1), r1(eb2), r1(eb3), r1(eg), r1(ebt))
    npad = -(-n // (8 * _NS)) * (8 * _NS)
    parts = _sc_scatter(e_new.reshape(_NW * k, _C, H), dstc,
                        jnp.zeros((npad, H), _f32))
    x_new = _node_mlp(x, parts, w1x, w1a, nw2, nw3,
                      r1(nb1), r1(nb2), r1(nb3), r1(ng), r1(nbt))
    return (x_new, e_new)


# speedup vs baseline: 1.1098x; 1.1098x over previous
import functools

import jax
import jax.numpy as jnp
from jax import lax
from jax.experimental import pallas as pl
from jax.experimental.pallas import tpu as pltpu
from jax.experimental.pallas import tpu_sc as plsc

H = 128
_NC = 2
_NS = 16
_NW = _NC * _NS
_C = 125
_L = 16

_f32 = jnp.float32


def _proj_body(x_ref, wd_ref, ws_ref, td_ref, ts_ref):
    xb = x_ref[...]
    td_ref[...] = jnp.dot(xb, wd_ref[...], preferred_element_type=_f32)
    ts_ref[...] = jnp.dot(xb, ws_ref[...], preferred_element_type=_f32)


def _proj(x, wd, ws, bn=1000):
    n = x.shape[0]
    return pl.pallas_call(
        _proj_body,
        grid=(n // bn,),
        in_specs=[
            pl.BlockSpec((bn, H), lambda i: (i, 0)),
            pl.BlockSpec((H, H), lambda i: (0, 0)),
            pl.BlockSpec((H, H), lambda i: (0, 0)),
        ],
        out_specs=[
            pl.BlockSpec((bn, H), lambda i: (i, 0)),
            pl.BlockSpec((bn, H), lambda i: (i, 0)),
        ],
        out_shape=[jax.ShapeDtypeStruct((n, H), _f32)] * 2,
    )(x, wd, ws)


def _sc_gather_sum(td, ts, dstc, srcc):
    k = dstc.shape[1]
    nch = _NW * k
    mesh = plsc.VectorSubcoreMesh(core_axis_name="c", subcore_axis_name="s",
                                  num_cores=_NC, num_subcores=_NS)

    @functools.partial(
        pl.kernel,
        out_type=jax.ShapeDtypeStruct((nch, _C, H), _f32),
        mesh=mesh,
        scratch_types=[
            pltpu.VMEM((k, _C), jnp.int32),
            pltpu.VMEM((k, _C), jnp.int32),
            pltpu.VMEM((2, _C, H), _f32),
            pltpu.VMEM((2, _C, H), _f32),
            pltpu.SemaphoreType.DMA,
            pltpu.SemaphoreType.DMA,
            pltpu.SemaphoreType.DMA,
            pltpu.SemaphoreType.DMA,
        ],
    )
    def run(td_h, ts_h, di_h, si_h, out_h, di_v, si_v, bd_v, bs_v,
            sg0, sg1, sw0, sw1):
        wid = lax.axis_index("s") * _NC + lax.axis_index("c")
        pltpu.sync_copy(di_h.at[wid], di_v)
        pltpu.sync_copy(si_h.at[wid], si_v)
        sgs = (sg0, sg1)
        sws = (sw0, sw1)

        def issue(j, slot):
            pltpu.async_copy(td_h.at[di_v.at[j]], bd_v.at[slot], sgs[slot])
            pltpu.async_copy(ts_h.at[si_v.at[j]], bs_v.at[slot], sgs[slot])

        def wait_g(slot):
            pltpu.make_async_copy(out_h.at[0], bd_v.at[slot], sgs[slot]).wait()
            pltpu.make_async_copy(out_h.at[0], bs_v.at[slot], sgs[slot]).wait()

        def drain_w(slot):
            pltpu.make_async_copy(bd_v.at[slot], out_h.at[0], sws[slot]).wait()

        def add_chunk(slot):
            def rbody(r, carry):
                for c8 in range(H // _L):
                    sl = pl.ds(c8 * _L, _L)
                    plsc.addupdate(bd_v.at[slot, r, sl], bs_v[slot, r, sl])
                return carry
            lax.fori_loop(0, _C, rbody, 0)

        def step(j, slot):
            wait_g(slot)
            add_chunk(slot)
            pltpu.async_copy(bd_v.at[slot], out_h.at[wid * k + j], sws[slot])

        issue(0, 0)

        def pair(t, carry):
            j0 = 2 * t

            @pl.when(t > 0)
            def _():
                drain_w(1)

            issue(j0 + 1, 1)
            step(j0, 0)

            @pl.when(j0 + 2 < k)
            def _():
                drain_w(0)
                issue(j0 + 2, 0)

            step(j0 + 1, 1)
            return carry

        lax.fori_loop(0, k // 2, pair, 0)
        drain_w(0)
        drain_w(1)

    return run(td, ts, dstc, srcc)


def _edge_body(gsum_ref, e_ref, we_ref, w2_ref, w3_ref,
               b1_ref, b2_ref, b3_ref, g_ref, bt_ref, out_ref):
    h = (gsum_ref[...]
         + jnp.dot(e_ref[...], we_ref[...], preferred_element_type=_f32)
         + b1_ref[...])
    h = jnp.maximum(h, 0.0)
    h = jnp.maximum(
        jnp.dot(h, w2_ref[...], preferred_element_type=_f32) + b2_ref[...], 0.0)
    h = jnp.dot(h, w3_ref[...], preferred_element_type=_f32) + b3_ref[...]
    m = jnp.mean(h, axis=-1, keepdims=True)
    c = h - m
    v = jnp.mean(c * c, axis=-1, keepdims=True)
    out_ref[...] = c * lax.rsqrt(v + 1e-5) * g_ref[...] + bt_ref[...]


def _edge_mlp(gsum, e, we, w2, w3, b1, b2, b3, g, bt, be=8000):
    ne = e.shape[0]
    wspec = pl.BlockSpec((H, H), lambda i: (0, 0))
    bspec = pl.BlockSpec((1, H), lambda i: (0, 0))
    blk = pl.BlockSpec((be, H), lambda i: (i, 0))
    return pl.pallas_call(
        _edge_body,
        grid=(ne // be,),
        in_specs=[blk, blk, wspec, wspec, wspec,
                  bspec, bspec, bspec, bspec, bspec],
        out_specs=blk,
        out_shape=jax.ShapeDtypeStruct((ne, H), _f32),
    )(gsum, e, we, w2, w3, b1, b2, b3, g, bt)


def _sc_scatter(enew, dstc, zeros):
    k = dstc.shape[1]
    n = zeros.shape[0]
    rpt = n // _NS
    mesh = plsc.VectorSubcoreMesh(core_axis_name="c", subcore_axis_name="s",
                                  num_cores=_NC, num_subcores=_NS)

    @functools.partial(
        pl.kernel,
        out_type=jax.ShapeDtypeStruct((_NC, n, H), _f32),
        mesh=mesh,
        scratch_types=[
            pltpu.VMEM((k, _C), jnp.int32),
            pltpu.VMEM((2, _C, H), _f32),
            pltpu.MemorySpace.VMEM_SHARED((n, H), _f32),
            pltpu.SemaphoreType.DMA,
            pltpu.SemaphoreType.DMA,
        ],
    )
    def run(en_h, di_h, z_h, out_h, di_v, buf_v, acc_s, sl0, sl1):
        cid = lax.axis_index("c")
        sid = lax.axis_index("s")
        wid = sid * _NC + cid
        row0 = sid * rpt
        pltpu.sync_copy(z_h.at[pl.ds(row0, rpt)], acc_s.at[pl.ds(row0, rpt)])
        plsc.subcore_barrier()
        pltpu.sync_copy(di_h.at[wid], di_v)
        sls = (sl0, sl1)

        def load(j, slot):
            pltpu.async_copy(en_h.at[wid * k + j], buf_v.at[slot], sls[slot])

        def wait_l(slot):
            pltpu.make_async_copy(en_h.at[0], buf_v.at[slot], sls[slot]).wait()

        load(0, 0)

        def pair(t, carry):
            j0 = 2 * t
            load(j0 + 1, 1)
            wait_l(0)
            pltpu.sync_copy(buf_v.at[0], acc_s.at[di_v.at[j0]], add=True)

            @pl.when(j0 + 2 < k)
            def _():
                load(j0 + 2, 0)

            wait_l(1)
            pltpu.sync_copy(buf_v.at[1], acc_s.at[di_v.at[j0 + 1]], add=True)
            return carry

        lax.fori_loop(0, k // 2, pair, 0)
        plsc.subcore_barrier()
        pltpu.sync_copy(acc_s.at[pl.ds(row0, rpt)],
                        out_h.at[cid, pl.ds(row0, rpt)])

    return run(enew, dstc, zeros)


def _node_body(x_ref, p_ref, w1x_ref, w1a_ref, w2_ref, w3_ref,
               b1_ref, b2_ref, b3_ref, g_ref, bt_ref, out_ref):
    xb = x_ref[...]
    agg = p_ref[0] + p_ref[1]
    z = (jnp.dot(xb, w1x_ref[...], preferred_element_type=_f32)
         + jnp.dot(agg, w1a_ref[...], preferred_element_type=_f32)
         + b1_ref[...])
    z = jnp.maximum(z, 0.0)
    z = jnp.maximum(
        jnp.dot(z, w2_ref[...], preferred_element_type=_f32) + b2_ref[...], 0.0)
    z = jnp.dot(z, w3_ref[...], preferred_element_type=_f32) + b3_ref[...]
    m = jnp.mean(z, axis=-1, keepdims=True)
    c = z - m
    v = jnp.mean(c * c, axis=-1, keepdims=True)
    out_ref[...] = xb + c * lax.rsqrt(v + 1e-5) * g_ref[...] + bt_ref[...]


def _node_mlp(x, parts, w1x, w1a, w2, w3, b1, b2, b3, g, bt, bn=1000):
    n = x.shape[0]
    wspec = pl.BlockSpec((H, H), lambda i: (0, 0))
    bspec = pl.BlockSpec((1, H), lambda i: (0, 0))
    return pl.pallas_call(
        _node_body,
        grid=(n // bn,),
        in_specs=[
            pl.BlockSpec((bn, H), lambda i: (i, 0)),
            pl.BlockSpec((_NC, bn, H), lambda i: (0, i, 0)),
            wspec, wspec, wspec, wspec,
            bspec, bspec, bspec, bspec, bspec,
        ],
        out_specs=pl.BlockSpec((bn, H), lambda i: (i, 0)),
        out_shape=jax.ShapeDtypeStruct((n, H), _f32),
    )(x, parts, w1x, w1a, w2, w3, b1, b2, b3, g, bt)


def kernel(x, edge_index, e, ew1, eb1, ew2, eb2, ew3, eb3, eg, ebt,
           nw1, nb1, nw2, nb2, nw3, nb3, ng, nbt):
    n = x.shape[0]
    ne = e.shape[0]
    k = ne // (_NW * _C)

    wd, ws, we = ew1[0:H], ew1[H:2 * H], ew1[2 * H:3 * H]
    w1x, w1a = nw1[0:H], nw1[H:2 * H]
    r1 = lambda v: v.reshape(1, H)

    srcc = edge_index[0].reshape(_NW, k, _C)
    dstc = edge_index[1].reshape(_NW, k, _C)

    td, ts = _proj(x, wd, ws)
    gsum = _sc_gather_sum(td, ts, dstc, srcc)
    e_new = _edge_mlp(gsum.reshape(ne, H), e,
                      we, ew2, ew3, r1(eb1), r1(eb2), r1(eb3), r1(eg), r1(ebt))
    npad = -(-n // (8 * _NS)) * (8 * _NS)
    parts = _sc_scatter(e_new.reshape(_NW * k, _C, H), dstc,
                        jnp.zeros((npad, H), _f32))
    x_new = _node_mlp(x, parts, w1x, w1a, nw2, nw3,
                      r1(nb1), r1(nb2), r1(nb3), r1(ng), r1(nbt))
    return (x_new, e_new)


# speedup vs baseline: 1.1172x; 1.0066x over previous
import functools

import jax
import jax.numpy as jnp
from jax import lax
from jax.experimental import pallas as pl
from jax.experimental.pallas import tpu as pltpu
from jax.experimental.pallas import tpu_sc as plsc

H = 128
_NC = 2
_NS = 16
_NW = _NC * _NS
_C = 125
_L = 16

_f32 = jnp.float32


def _proj_body(x_ref, wd_ref, ws_ref, td_ref, ts_ref):
    xb = x_ref[...]
    td_ref[...] = jnp.dot(xb, wd_ref[...], preferred_element_type=_f32)
    ts_ref[...] = jnp.dot(xb, ws_ref[...], preferred_element_type=_f32)


def _proj(x, wd, ws, bn=1000):
    n = x.shape[0]
    return pl.pallas_call(
        _proj_body,
        grid=(n // bn,),
        in_specs=[
            pl.BlockSpec((bn, H), lambda i: (i, 0)),
            pl.BlockSpec((H, H), lambda i: (0, 0)),
            pl.BlockSpec((H, H), lambda i: (0, 0)),
        ],
        out_specs=[
            pl.BlockSpec((bn, H), lambda i: (i, 0)),
            pl.BlockSpec((bn, H), lambda i: (i, 0)),
        ],
        out_shape=[jax.ShapeDtypeStruct((n, H), _f32)] * 2,
    )(x, wd, ws)


def _sc_gather_sum(td, ts, dstc, srcc):
    k = dstc.shape[1]
    nch = _NW * k
    mesh = plsc.VectorSubcoreMesh(core_axis_name="c", subcore_axis_name="s",
                                  num_cores=_NC, num_subcores=_NS)

    @functools.partial(
        pl.kernel,
        out_type=jax.ShapeDtypeStruct((nch, _C, H), _f32),
        mesh=mesh,
        scratch_types=[
            pltpu.VMEM((k, _C), jnp.int32),
            pltpu.VMEM((k, _C), jnp.int32),
            pltpu.VMEM((2, _C, H), _f32),
            pltpu.VMEM((2, _C, H), _f32),
            pltpu.SemaphoreType.DMA,
            pltpu.SemaphoreType.DMA,
            pltpu.SemaphoreType.DMA,
            pltpu.SemaphoreType.DMA,
        ],
    )
    def run(td_h, ts_h, di_h, si_h, out_h, di_v, si_v, bd_v, bs_v,
            sg0, sg1, sw0, sw1):
        wid = lax.axis_index("s") * _NC + lax.axis_index("c")
        pltpu.sync_copy(di_h.at[wid], di_v)
        pltpu.sync_copy(si_h.at[wid], si_v)
        sgs = (sg0, sg1)
        sws = (sw0, sw1)

        def issue(j, slot):
            pltpu.async_copy(td_h.at[di_v.at[j]], bd_v.at[slot], sgs[slot])
            pltpu.async_copy(ts_h.at[si_v.at[j]], bs_v.at[slot], sgs[slot])

        def wait_g(slot):
            pltpu.make_async_copy(out_h.at[0], bd_v.at[slot], sgs[slot]).wait()
            pltpu.make_async_copy(out_h.at[0], bs_v.at[slot], sgs[slot]).wait()

        def drain_w(slot):
            pltpu.make_async_copy(bd_v.at[slot], out_h.at[0], sws[slot]).wait()

        def add_chunk(slot):
            def rbody(r, carry):
                for c8 in range(H // _L):
                    sl = pl.ds(c8 * _L, _L)
                    plsc.addupdate(bd_v.at[slot, r, sl], bs_v[slot, r, sl])
                return carry
            lax.fori_loop(0, _C, rbody, 0)

        def step(j, slot):
            wait_g(slot)
            add_chunk(slot)
            pltpu.async_copy(bd_v.at[slot], out_h.at[wid * k + j], sws[slot])

        issue(0, 0)

        def pair(t, carry):
            j0 = 2 * t

            @pl.when(t > 0)
            def _():
                drain_w(1)

            issue(j0 + 1, 1)
            step(j0, 0)

            @pl.when(j0 + 2 < k)
            def _():
                drain_w(0)
                issue(j0 + 2, 0)

            step(j0 + 1, 1)
            return carry

        lax.fori_loop(0, k // 2, pair, 0)
        drain_w(0)
        drain_w(1)

    return run(td, ts, dstc, srcc)


def _edge_body(gsum_ref, e_ref, we_ref, w2_ref, w3_ref,
               b1_ref, b2_ref, b3_ref, g_ref, bt_ref, out_ref):
    h = (gsum_ref[...]
         + jnp.dot(e_ref[...], we_ref[...], preferred_element_type=_f32)
         + b1_ref[...])
    h = jnp.maximum(h, 0.0)
    h = jnp.maximum(
        jnp.dot(h, w2_ref[...], preferred_element_type=_f32) + b2_ref[...], 0.0)
    h = jnp.dot(h, w3_ref[...], preferred_element_type=_f32) + b3_ref[...]
    m = jnp.mean(h, axis=-1, keepdims=True)
    c = h - m
    v = jnp.mean(c * c, axis=-1, keepdims=True)
    out_ref[...] = c * lax.rsqrt(v + 1e-5) * g_ref[...] + bt_ref[...]


def _edge_mlp(gsum, e, we, w2, w3, b1, b2, b3, g, bt, be=16000):
    ne = e.shape[0]
    wspec = pl.BlockSpec((H, H), lambda i: (0, 0))
    bspec = pl.BlockSpec((1, H), lambda i: (0, 0))
    blk = pl.BlockSpec((be, H), lambda i: (i, 0))
    return pl.pallas_call(
        _edge_body,
        grid=(ne // be,),
        in_specs=[blk, blk, wspec, wspec, wspec,
                  bspec, bspec, bspec, bspec, bspec],
        out_specs=blk,
        out_shape=jax.ShapeDtypeStruct((ne, H), _f32),
    )(gsum, e, we, w2, w3, b1, b2, b3, g, bt)


def _sc_scatter(enew, dstc, zeros):
    k = dstc.shape[1]
    n = zeros.shape[0]
    rpt = n // _NS
    mesh = plsc.VectorSubcoreMesh(core_axis_name="c", subcore_axis_name="s",
                                  num_cores=_NC, num_subcores=_NS)

    @functools.partial(
        pl.kernel,
        out_type=jax.ShapeDtypeStruct((_NC, n, H), _f32),
        mesh=mesh,
        scratch_types=[
            pltpu.VMEM((k, _C), jnp.int32),
            pltpu.VMEM((2, _C, H), _f32),
            pltpu.MemorySpace.VMEM_SHARED((n, H), _f32),
            pltpu.SemaphoreType.DMA,
            pltpu.SemaphoreType.DMA,
        ],
    )
    def run(en_h, di_h, z_h, out_h, di_v, buf_v, acc_s, sl0, sl1):
        cid = lax.axis_index("c")
        sid = lax.axis_index("s")
        wid = sid * _NC + cid
        row0 = sid * rpt
        pltpu.sync_copy(z_h.at[pl.ds(row0, rpt)], acc_s.at[pl.ds(row0, rpt)])
        plsc.subcore_barrier()
        pltpu.sync_copy(di_h.at[wid], di_v)
        sls = (sl0, sl1)

        def load(j, slot):
            pltpu.async_copy(en_h.at[wid * k + j], buf_v.at[slot], sls[slot])

        def wait_l(slot):
            pltpu.make_async_copy(en_h.at[0], buf_v.at[slot], sls[slot]).wait()

        load(0, 0)

        def pair(t, carry):
            j0 = 2 * t
            load(j0 + 1, 1)
            wait_l(0)
            pltpu.sync_copy(buf_v.at[0], acc_s.at[di_v.at[j0]], add=True)

            @pl.when(j0 + 2 < k)
            def _():
                load(j0 + 2, 0)

            wait_l(1)
            pltpu.sync_copy(buf_v.at[1], acc_s.at[di_v.at[j0 + 1]], add=True)
            return carry

        lax.fori_loop(0, k // 2, pair, 0)
        plsc.subcore_barrier()
        pltpu.sync_copy(acc_s.at[pl.ds(row0, rpt)],
                        out_h.at[cid, pl.ds(row0, rpt)])

    return run(enew, dstc, zeros)


def _node_body(x_ref, p_ref, w1x_ref, w1a_ref, w2_ref, w3_ref,
               b1_ref, b2_ref, b3_ref, g_ref, bt_ref, out_ref):
    xb = x_ref[...]
    agg = p_ref[0] + p_ref[1]
    z = (jnp.dot(xb, w1x_ref[...], preferred_element_type=_f32)
         + jnp.dot(agg, w1a_ref[...], preferred_element_type=_f32)
         + b1_ref[...])
    z = jnp.maximum(z, 0.0)
    z = jnp.maximum(
        jnp.dot(z, w2_ref[...], preferred_element_type=_f32) + b2_ref[...], 0.0)
    z = jnp.dot(z, w3_ref[...], preferred_element_type=_f32) + b3_ref[...]
    m = jnp.mean(z, axis=-1, keepdims=True)
    c = z - m
    v = jnp.mean(c * c, axis=-1, keepdims=True)
    out_ref[...] = xb + c * lax.rsqrt(v + 1e-5) * g_ref[...] + bt_ref[...]


def _node_mlp(x, parts, w1x, w1a, w2, w3, b1, b2, b3, g, bt, bn=1000):
    n = x.shape[0]
    wspec = pl.BlockSpec((H, H), lambda i: (0, 0))
    bspec = pl.BlockSpec((1, H), lambda i: (0, 0))
    return pl.pallas_call(
        _node_body,
        grid=(n // bn,),
        in_specs=[
            pl.BlockSpec((bn, H), lambda i: (i, 0)),
            pl.BlockSpec((_NC, bn, H), lambda i: (0, i, 0)),
            wspec, wspec, wspec, wspec,
            bspec, bspec, bspec, bspec, bspec,
        ],
        out_specs=pl.BlockSpec((bn, H), lambda i: (i, 0)),
        out_shape=jax.ShapeDtypeStruct((n, H), _f32),
    )(x, parts, w1x, w1a, w2, w3, b1, b2, b3, g, bt)


def kernel(x, edge_index, e, ew1, eb1, ew2, eb2, ew3, eb3, eg, ebt,
           nw1, nb1, nw2, nb2, nw3, nb3, ng, nbt):
    n = x.shape[0]
    ne = e.shape[0]
    k = ne // (_NW * _C)

    wd, ws, we = ew1[0:H], ew1[H:2 * H], ew1[2 * H:3 * H]
    w1x, w1a = nw1[0:H], nw1[H:2 * H]
    r1 = lambda v: v.reshape(1, H)

    srcc = edge_index[0].reshape(_NW, k, _C)
    dstc = edge_index[1].reshape(_NW, k, _C)

    td, ts = _proj(x, wd, ws)
    gsum = _sc_gather_sum(td, ts, dstc, srcc)
    e_new = _edge_mlp(gsum.reshape(ne, H), e,
                      we, ew2, ew3, r1(eb1), r1(eb2), r1(eb3), r1(eg), r1(ebt))
    npad = -(-n // (8 * _NS)) * (8 * _NS)
    parts = _sc_scatter(e_new.reshape(_NW * k, _C, H), dstc,
                        jnp.zeros((npad, H), _f32))
    x_new = _node_mlp(x, parts, w1x, w1a, nw2, nw3,
                      r1(nb1), r1(nb2), r1(nb3), r1(ng), r1(nbt))
    return (x_new, e_new)


# speedup vs baseline: 1.5757x; 1.4104x over previous
import functools

import jax
import jax.numpy as jnp
from jax import lax
from jax.experimental import pallas as pl
from jax.experimental.pallas import tpu as pltpu
from jax.experimental.pallas import tpu_sc as plsc

H = 128
_NC = 2
_NS = 16
_NW = _NC * _NS
_C = 80
_L = 16

_f32 = jnp.float32


def _proj_body(x_ref, wd_ref, ws_ref, td_ref, ts_ref):
    xb = x_ref[...]
    td_ref[...] = jnp.dot(xb, wd_ref[...], preferred_element_type=_f32)
    ts_ref[...] = jnp.dot(xb, ws_ref[...], preferred_element_type=_f32)


def _proj(x, wd, ws, bn=1000):
    n = x.shape[0]
    return pl.pallas_call(
        _proj_body,
        grid=(n // bn,),
        in_specs=[
            pl.BlockSpec((bn, H), lambda i: (i, 0)),
            pl.BlockSpec((H, H), lambda i: (0, 0)),
            pl.BlockSpec((H, H), lambda i: (0, 0)),
        ],
        out_specs=[
            pl.BlockSpec((bn, H), lambda i: (i, 0)),
            pl.BlockSpec((bn, H), lambda i: (i, 0)),
        ],
        out_shape=[jax.ShapeDtypeStruct((n, H), _f32)] * 2,
    )(x, wd, ws)


def _sc_gather_sum(td, ts, dstc, srcc):
    k = dstc.shape[1]
    epw = k * _C
    mesh = plsc.VectorSubcoreMesh(core_axis_name="c", subcore_axis_name="s",
                                  num_cores=_NC, num_subcores=_NS)

    @functools.partial(
        pl.kernel,
        out_type=jax.ShapeDtypeStruct((_NW * epw, H), _f32),
        mesh=mesh,
        scratch_types=[
            pltpu.VMEM((k, _C), jnp.int32),
            pltpu.VMEM((k, _C), jnp.int32),
            pltpu.VMEM((2, _C, H), _f32),
            pltpu.VMEM((2, _C, H), _f32),
            pltpu.SemaphoreType.DMA,
            pltpu.SemaphoreType.DMA,
            pltpu.SemaphoreType.DMA,
            pltpu.SemaphoreType.DMA,
        ],
    )
    def run(td_h, ts_h, di_h, si_h, out_h, di_v, si_v, bd_v, bs_v,
            sg0, sg1, sw0, sw1):
        wid = lax.axis_index("s") * _NC + lax.axis_index("c")
        row0 = wid * epw
        pltpu.sync_copy(di_h.at[wid], di_v)
        pltpu.sync_copy(si_h.at[wid], si_v)
        sgs = (sg0, sg1)
        sws = (sw0, sw1)

        def issue(j, slot):
            pltpu.async_copy(td_h.at[di_v.at[j]], bd_v.at[slot], sgs[slot])
            pltpu.async_copy(ts_h.at[si_v.at[j]], bs_v.at[slot], sgs[slot])

        def wait_g(slot):
            pltpu.make_async_copy(out_h.at[pl.ds(0, _C)], bd_v.at[slot],
                                  sgs[slot]).wait()
            pltpu.make_async_copy(out_h.at[pl.ds(0, _C)], bs_v.at[slot],
                                  sgs[slot]).wait()

        def drain_w(slot):
            pltpu.make_async_copy(bd_v.at[slot], out_h.at[pl.ds(0, _C)],
                                  sws[slot]).wait()

        def add_chunk(slot):
            def rbody(r, carry):
                for c8 in range(H // _L):
                    sl = pl.ds(c8 * _L, _L)
                    plsc.addupdate(bd_v.at[slot, r, sl], bs_v[slot, r, sl])
                return carry
            lax.fori_loop(0, _C, rbody, 0)

        def step(j, slot):
            wait_g(slot)
            add_chunk(slot)
            pltpu.async_copy(bd_v.at[slot],
                             out_h.at[pl.ds(row0 + j * _C, _C)], sws[slot])

        issue(0, 0)

        def pair(t, carry):
            j0 = 2 * t

            @pl.when(t > 0)
            def _():
                drain_w(1)

            issue(j0 + 1, 1)
            step(j0, 0)

            @pl.when(j0 + 2 < k)
            def _():
                drain_w(0)
                issue(j0 + 2, 0)

            step(j0 + 1, 1)
            return carry

        lax.fori_loop(0, k // 2, pair, 0)
        if k % 2 == 1:
            step(k - 1, 0)
        drain_w(0)
        drain_w(1)

    return run(td, ts, dstc, srcc)


def _edge_body(gsum_ref, e_ref, we_ref, w2_ref, w3_ref,
               b1_ref, b2_ref, b3_ref, g_ref, bt_ref, out_ref):
    h = (gsum_ref[...]
         + jnp.dot(e_ref[...], we_ref[...], preferred_element_type=_f32)
         + b1_ref[...])
    h = jnp.maximum(h, 0.0)
    h = jnp.maximum(
        jnp.dot(h, w2_ref[...], preferred_element_type=_f32) + b2_ref[...], 0.0)
    h = jnp.dot(h, w3_ref[...], preferred_element_type=_f32) + b3_ref[...]
    m = jnp.mean(h, axis=-1, keepdims=True)
    c = h - m
    v = jnp.mean(c * c, axis=-1, keepdims=True)
    out_ref[...] = c * lax.rsqrt(v + 1e-5) * g_ref[...] + bt_ref[...]


def _edge_mlp(gsum, e, we, w2, w3, b1, b2, b3, g, bt, be=16000):
    ne = e.shape[0]
    wspec = pl.BlockSpec((H, H), lambda i: (0, 0))
    bspec = pl.BlockSpec((1, H), lambda i: (0, 0))
    blk = pl.BlockSpec((be, H), lambda i: (i, 0))
    return pl.pallas_call(
        _edge_body,
        grid=(ne // be,),
        in_specs=[blk, blk, wspec, wspec, wspec,
                  bspec, bspec, bspec, bspec, bspec],
        out_specs=blk,
        out_shape=jax.ShapeDtypeStruct((ne, H), _f32),
    )(gsum, e, we, w2, w3, b1, b2, b3, g, bt)


def _sc_scatter(enew, dstc, zeros):
    k = dstc.shape[1]
    epw = k * _C
    n = zeros.shape[0]
    rpt = n // _NS
    mesh = plsc.VectorSubcoreMesh(core_axis_name="c", subcore_axis_name="s",
                                  num_cores=_NC, num_subcores=_NS)

    @functools.partial(
        pl.kernel,
        out_type=jax.ShapeDtypeStruct((_NC, n, H), _f32),
        mesh=mesh,
        scratch_types=[
            pltpu.VMEM((k, _C), jnp.int32),
            pltpu.VMEM((2, _C, H), _f32),
            pltpu.MemorySpace.VMEM_SHARED((n, H), _f32),
            pltpu.SemaphoreType.DMA,
            pltpu.SemaphoreType.DMA,
        ],
    )
    def run(en_h, di_h, z_h, out_h, di_v, buf_v, acc_s, sl0, sl1):
        cid = lax.axis_index("c")
        sid = lax.axis_index("s")
        wid = sid * _NC + cid
        arow0 = sid * rpt
        erow0 = wid * epw
        pltpu.sync_copy(z_h.at[pl.ds(arow0, rpt)], acc_s.at[pl.ds(arow0, rpt)])
        plsc.subcore_barrier()
        pltpu.sync_copy(di_h.at[wid], di_v)
        sls = (sl0, sl1)

        def load(j, slot):
            pltpu.async_copy(en_h.at[pl.ds(erow0 + j * _C, _C)],
                             buf_v.at[slot], sls[slot])

        def wait_l(slot):
            pltpu.make_async_copy(en_h.at[pl.ds(0, _C)], buf_v.at[slot],
                                  sls[slot]).wait()

        def scat(j, slot):
            wait_l(slot)
            pltpu.sync_copy(buf_v.at[slot], acc_s.at[di_v.at[j]], add=True)

        load(0, 0)

        def pair(t, carry):
            j0 = 2 * t
            load(j0 + 1, 1)
            scat(j0, 0)

            @pl.when(j0 + 2 < k)
            def _():
                load(j0 + 2, 0)

            scat(j0 + 1, 1)
            return carry

        lax.fori_loop(0, k // 2, pair, 0)
        if k % 2 == 1:
            scat(k - 1, 0)
        plsc.subcore_barrier()
        pltpu.sync_copy(acc_s.at[pl.ds(arow0, rpt)],
                        out_h.at[cid, pl.ds(arow0, rpt)])

    return run(enew, dstc, zeros)


def _node_body(x_ref, p_ref, w1x_ref, w1a_ref, w2_ref, w3_ref,
               b1_ref, b2_ref, b3_ref, g_ref, bt_ref, out_ref):
    xb = x_ref[...]
    agg = p_ref[0] + p_ref[1]
    z = (jnp.dot(xb, w1x_ref[...], preferred_element_type=_f32)
         + jnp.dot(agg, w1a_ref[...], preferred_element_type=_f32)
         + b1_ref[...])
    z = jnp.maximum(z, 0.0)
    z = jnp.maximum(
        jnp.dot(z, w2_ref[...], preferred_element_type=_f32) + b2_ref[...], 0.0)
    z = jnp.dot(z, w3_ref[...], preferred_element_type=_f32) + b3_ref[...]
    m = jnp.mean(z, axis=-1, keepdims=True)
    c = z - m
    v = jnp.mean(c * c, axis=-1, keepdims=True)
    out_ref[...] = xb + c * lax.rsqrt(v + 1e-5) * g_ref[...] + bt_ref[...]


def _node_mlp(x, parts, w1x, w1a, w2, w3, b1, b2, b3, g, bt, bn=1000):
    n = x.shape[0]
    wspec = pl.BlockSpec((H, H), lambda i: (0, 0))
    bspec = pl.BlockSpec((1, H), lambda i: (0, 0))
    return pl.pallas_call(
        _node_body,
        grid=(n // bn,),
        in_specs=[
            pl.BlockSpec((bn, H), lambda i: (i, 0)),
            pl.BlockSpec((_NC, bn, H), lambda i: (0, i, 0)),
            wspec, wspec, wspec, wspec,
            bspec, bspec, bspec, bspec, bspec,
        ],
        out_specs=pl.BlockSpec((bn, H), lambda i: (i, 0)),
        out_shape=jax.ShapeDtypeStruct((n, H), _f32),
    )(x, parts, w1x, w1a, w2, w3, b1, b2, b3, g, bt)


def kernel(x, edge_index, e, ew1, eb1, ew2, eb2, ew3, eb3, eg, ebt,
           nw1, nb1, nw2, nb2, nw3, nb3, ng, nbt):
    n = x.shape[0]
    ne = e.shape[0]
    k = ne // (_NW * _C)

    wd, ws, we = ew1[0:H], ew1[H:2 * H], ew1[2 * H:3 * H]
    w1x, w1a = nw1[0:H], nw1[H:2 * H]
    r1 = lambda v: v.reshape(1, H)

    srcc = edge_index[0].reshape(_NW, k, _C)
    dstc = edge_index[1].reshape(_NW, k, _C)

    td, ts = _proj(x, wd, ws)
    gsum = _sc_gather_sum(td, ts, dstc, srcc)
    e_new = _edge_mlp(gsum, e,
                      we, ew2, ew3, r1(eb1), r1(eb2), r1(eb3), r1(eg), r1(ebt))
    npad = -(-n // (8 * _NS)) * (8 * _NS)
    parts = _sc_scatter(e_new, dstc, jnp.zeros((npad, H), _f32))
    x_new = _node_mlp(x, parts, w1x, w1a, nw2, nw3,
                      r1(nb1), r1(nb2), r1(nb3), r1(ng), r1(nbt))
    return (x_new, e_new)


# speedup vs baseline: 1.6200x; 1.0281x over previous
import functools

import jax
import jax.numpy as jnp
from jax import lax
from jax.experimental import pallas as pl
from jax.experimental.pallas import tpu as pltpu
from jax.experimental.pallas import tpu_sc as plsc

H = 128
_NC = 2
_NS = 16
_NW = _NC * _NS
_C = 80
_L = 16

_f32 = jnp.float32


def _proj_body(x_ref, w_ref, td_ref, ts_ref):
    xb = x_ref[...]
    td_ref[...] = jnp.dot(xb, w_ref[0:H], preferred_element_type=_f32)
    ts_ref[...] = jnp.dot(xb, w_ref[H:2 * H], preferred_element_type=_f32)


def _proj(x, ew1, bn=1000):
    n = x.shape[0]
    return pl.pallas_call(
        _proj_body,
        grid=(n // bn,),
        in_specs=[
            pl.BlockSpec((bn, H), lambda i: (i, 0)),
            pl.BlockSpec((3 * H, H), lambda i: (0, 0)),
        ],
        out_specs=[
            pl.BlockSpec((bn, H), lambda i: (i, 0)),
            pl.BlockSpec((bn, H), lambda i: (i, 0)),
        ],
        out_shape=[jax.ShapeDtypeStruct((n, H), _f32)] * 2,
    )(x, ew1)


def _sc_gather_sum(td, ts, eic):
    k = eic.shape[2]
    epw = k * _C
    mesh = plsc.VectorSubcoreMesh(core_axis_name="c", subcore_axis_name="s",
                                  num_cores=_NC, num_subcores=_NS)

    @functools.partial(
        pl.kernel,
        out_type=jax.ShapeDtypeStruct((_NW * epw, H), _f32),
        mesh=mesh,
        scratch_types=[
            pltpu.VMEM((k, _C), jnp.int32),
            pltpu.VMEM((k, _C), jnp.int32),
            pltpu.VMEM((2, _C, H), _f32),
            pltpu.VMEM((2, _C, H), _f32),
            pltpu.SemaphoreType.DMA,
            pltpu.SemaphoreType.DMA,
            pltpu.SemaphoreType.DMA,
            pltpu.SemaphoreType.DMA,
        ],
    )
    def run(td_h, ts_h, ei_h, out_h, di_v, si_v, bd_v, bs_v,
            sg0, sg1, sw0, sw1):
        wid = lax.axis_index("s") * _NC + lax.axis_index("c")
        row0 = wid * epw
        pltpu.sync_copy(ei_h.at[1, wid], di_v)
        pltpu.sync_copy(ei_h.at[0, wid], si_v)
        sgs = (sg0, sg1)
        sws = (sw0, sw1)

        def issue(j, slot):
            pltpu.async_copy(td_h.at[di_v.at[j]], bd_v.at[slot], sgs[slot])
            pltpu.async_copy(ts_h.at[si_v.at[j]], bs_v.at[slot], sgs[slot])

        def wait_g(slot):
            pltpu.make_async_copy(out_h.at[pl.ds(0, _C)], bd_v.at[slot],
                                  sgs[slot]).wait()
            pltpu.make_async_copy(out_h.at[pl.ds(0, _C)], bs_v.at[slot],
                                  sgs[slot]).wait()

        def drain_w(slot):
            pltpu.make_async_copy(bd_v.at[slot], out_h.at[pl.ds(0, _C)],
                                  sws[slot]).wait()

        def add_chunk(slot):
            def rbody(r, carry):
                for c8 in range(H // _L):
                    sl = pl.ds(c8 * _L, _L)
                    plsc.addupdate(bd_v.at[slot, r, sl], bs_v[slot, r, sl])
                return carry
            lax.fori_loop(0, _C, rbody, 0)

        def step(j, slot):
            wait_g(slot)
            add_chunk(slot)
            pltpu.async_copy(bd_v.at[slot],
                             out_h.at[pl.ds(row0 + j * _C, _C)], sws[slot])

        issue(0, 0)

        def pair(t, carry):
            j0 = 2 * t

            @pl.when(t > 0)
            def _():
                drain_w(1)

            issue(j0 + 1, 1)
            step(j0, 0)

            @pl.when(j0 + 2 < k)
            def _():
                drain_w(0)
                issue(j0 + 2, 0)

            step(j0 + 1, 1)
            return carry

        lax.fori_loop(0, k // 2, pair, 0)
        if k % 2 == 1:
            step(k - 1, 0)
        drain_w(0)
        drain_w(1)

    return run(td, ts, eic)


def _edge_body(gsum_ref, e_ref, w1_ref, w2_ref, w3_ref,
               b1_ref, b2_ref, b3_ref, g_ref, bt_ref, out_ref):
    h = (gsum_ref[...]
         + jnp.dot(e_ref[...], w1_ref[2 * H:3 * H],
                   preferred_element_type=_f32)
         + b1_ref[...])
    h = jnp.maximum(h, 0.0)
    h = jnp.maximum(
        jnp.dot(h, w2_ref[...], preferred_element_type=_f32) + b2_ref[...], 0.0)
    h = jnp.dot(h, w3_ref[...], preferred_element_type=_f32) + b3_ref[...]
    m = jnp.mean(h, axis=-1, keepdims=True)
    c = h - m
    v = jnp.mean(c * c, axis=-1, keepdims=True)
    out_ref[...] = c * lax.rsqrt(v + 1e-5) * g_ref[...] + bt_ref[...]


def _edge_mlp(gsum, e, ew1, w2, w3, b1, b2, b3, g, bt, be=16000):
    ne = e.shape[0]
    wspec = pl.BlockSpec((H, H), lambda i: (0, 0))
    bspec = pl.BlockSpec((H,), lambda i: (0,))
    blk = pl.BlockSpec((be, H), lambda i: (i, 0))
    return pl.pallas_call(
        _edge_body,
        grid=(ne // be,),
        in_specs=[blk, blk, pl.BlockSpec((3 * H, H), lambda i: (0, 0)),
                  wspec, wspec, bspec, bspec, bspec, bspec, bspec],
        out_specs=blk,
        out_shape=jax.ShapeDtypeStruct((ne, H), _f32),
    )(gsum, e, ew1, w2, w3, b1, b2, b3, g, bt)


def _sc_scatter(enew, eic, zeros):
    k = eic.shape[2]
    epw = k * _C
    n = zeros.shape[0]
    rpt = n // _NS
    mesh = plsc.VectorSubcoreMesh(core_axis_name="c", subcore_axis_name="s",
                                  num_cores=_NC, num_subcores=_NS)

    @functools.partial(
        pl.kernel,
        out_type=jax.ShapeDtypeStruct((_NC, n, H), _f32),
        mesh=mesh,
        scratch_types=[
            pltpu.VMEM((k, _C), jnp.int32),
            pltpu.VMEM((2, _C, H), _f32),
            pltpu.MemorySpace.VMEM_SHARED((n, H), _f32),
            pltpu.SemaphoreType.DMA,
            pltpu.SemaphoreType.DMA,
        ],
    )
    def run(en_h, ei_h, z_h, out_h, di_v, buf_v, acc_s, sl0, sl1):
        cid = lax.axis_index("c")
        sid = lax.axis_index("s")
        wid = sid * _NC + cid
        arow0 = sid * rpt
        erow0 = wid * epw
        pltpu.sync_copy(z_h.at[pl.ds(arow0, rpt)], acc_s.at[pl.ds(arow0, rpt)])
        plsc.subcore_barrier()
        pltpu.sync_copy(ei_h.at[1, wid], di_v)
        sls = (sl0, sl1)

        def load(j, slot):
            pltpu.async_copy(en_h.at[pl.ds(erow0 + j * _C, _C)],
                             buf_v.at[slot], sls[slot])

        def wait_l(slot):
            pltpu.make_async_copy(en_h.at[pl.ds(0, _C)], buf_v.at[slot],
                                  sls[slot]).wait()

        def scat(j, slot):
            wait_l(slot)
            pltpu.sync_copy(buf_v.at[slot], acc_s.at[di_v.at[j]], add=True)

        load(0, 0)

        def pair(t, carry):
            j0 = 2 * t
            load(j0 + 1, 1)
            scat(j0, 0)

            @pl.when(j0 + 2 < k)
            def _():
                load(j0 + 2, 0)

            scat(j0 + 1, 1)
            return carry

        lax.fori_loop(0, k // 2, pair, 0)
        if k % 2 == 1:
            scat(k - 1, 0)
        plsc.subcore_barrier()
        pltpu.sync_copy(acc_s.at[pl.ds(arow0, rpt)],
                        out_h.at[cid, pl.ds(arow0, rpt)])

    return run(enew, eic, zeros)


def _node_body(x_ref, p_ref, w1_ref, w2_ref, w3_ref,
               b1_ref, b2_ref, b3_ref, g_ref, bt_ref, out_ref):
    xb = x_ref[...]
    agg = p_ref[0] + p_ref[1]
    z = (jnp.dot(xb, w1_ref[0:H], preferred_element_type=_f32)
         + jnp.dot(agg, w1_ref[H:2 * H], preferred_element_type=_f32)
         + b1_ref[...])
    z = jnp.maximum(z, 0.0)
    z = jnp.maximum(
        jnp.dot(z, w2_ref[...], preferred_element_type=_f32) + b2_ref[...], 0.0)
    z = jnp.dot(z, w3_ref[...], preferred_element_type=_f32) + b3_ref[...]
    m = jnp.mean(z, axis=-1, keepdims=True)
    c = z - m
    v = jnp.mean(c * c, axis=-1, keepdims=True)
    out_ref[...] = xb + c * lax.rsqrt(v + 1e-5) * g_ref[...] + bt_ref[...]


def _node_mlp(x, parts, nw1, w2, w3, b1, b2, b3, g, bt, bn=1000):
    n = x.shape[0]
    wspec = pl.BlockSpec((H, H), lambda i: (0, 0))
    bspec = pl.BlockSpec((H,), lambda i: (0,))
    return pl.pallas_call(
        _node_body,
        grid=(n // bn,),
        in_specs=[
            pl.BlockSpec((bn, H), lambda i: (i, 0)),
            pl.BlockSpec((_NC, bn, H), lambda i: (0, i, 0)),
            pl.BlockSpec((2 * H, H), lambda i: (0, 0)),
            wspec, wspec,
            bspec, bspec, bspec, bspec, bspec,
        ],
        out_specs=pl.BlockSpec((bn, H), lambda i: (i, 0)),
        out_shape=jax.ShapeDtypeStruct((n, H), _f32),
    )(x, parts, nw1, w2, w3, b1, b2, b3, g, bt)


def kernel(x, edge_index, e, ew1, eb1, ew2, eb2, ew3, eb3, eg, ebt,
           nw1, nb1, nw2, nb2, nw3, nb3, ng, nbt):
    n = x.shape[0]
    ne = e.shape[0]
    k = ne // (_NW * _C)

    eic = edge_index.reshape(2, _NW, k, _C)

    td, ts = _proj(x, ew1)
    gsum = _sc_gather_sum(td, ts, eic)
    e_new = _edge_mlp(gsum, e, ew1, ew2, ew3, eb1, eb2, eb3, eg, ebt)
    npad = -(-n // (8 * _NS)) * (8 * _NS)
    parts = _sc_scatter(e_new, eic, jnp.zeros((npad, H), _f32))
    x_new = _node_mlp(x, parts, nw1, nw2, nw3, nb1, nb2, nb3, ng, nbt)
    return (x_new, e_new)
